# Initial kernel scaffold; baseline (speedup 1.0000x reference)
import jax
import jax.numpy as jnp
from jax.experimental import pallas as pl


def kernel(x, edge_index, batch, W1, att_src1, att_dst1, b1, W2, att_src2, att_dst2, b2, Wc1, bc1, Wc2, bc2):
    raise NotImplementedError("write your pallas kernel here")


# speedup vs baseline: 23.3691x; 23.3691x over previous
import functools

import jax
import jax.numpy as jnp
from jax import lax
from jax.experimental import pallas as pl
from jax.experimental.pallas import tpu as pltpu
from jax.experimental.pallas import tpu_sc as plsc

N = 10000
E = 320000
D = 128
H = 128
G = 16

NP = 10240
K = 128
NCH = E // K
NT = 16
STRIPE = NP // NT
BLK = 1024

_SC_SCRATCH = lambda: [
    pltpu.VMEM((NP,), jnp.float32),
    pltpu.VMEM((NP,), jnp.float32),
    pltpu.VMEM((NP,), jnp.float32),
    pltpu.VMEM((K,), jnp.int32),
    pltpu.VMEM((K,), jnp.int32),
    pltpu.VMEM((K,), jnp.int32),
    pltpu.VMEM((K,), jnp.float32),
    pltpu.VMEM((K, H), jnp.float32),
    pltpu.VMEM((16,), jnp.float32),
    pltpu.VMEM_SHARED((NP, H), jnp.float32),
    pltpu.VMEM_SHARED((NP,), jnp.float32),
    pltpu.SemaphoreType.DMA,
]


def _alpha_groups(srcbuf, dstbuf, a_s_loc, a_d_loc, cv, den_loc, exbuf):
    for g in range(K // 16):
        sl = pl.ds(g * 16, 16)
        sv = srcbuf[sl]
        dv = dstbuf[sl]
        a = plsc.load_gather(a_s_loc, [sv]) + plsc.load_gather(a_d_loc, [dv])
        e = jnp.where(a > 0, a, 0.2 * a)
        ex = jnp.exp(e - cv)
        if den_loc is not None:
            den = plsc.load_gather(den_loc, [dv])
            ex = ex / (den + 1e-16)
        exbuf[sl] = ex


def _make_sc_layer(two_heads):
    @functools.partial(
        pl.kernel,
        mesh=plsc.VectorSubcoreMesh(core_axis_name="c", subcore_axis_name="s"),
        out_type=jax.ShapeDtypeStruct((2, NP, H), jnp.float32),
        scratch_types=_SC_SCRATCH(),
        compiler_params=pltpu.CompilerParams(needs_layout_passes=False),
    )
    def sc_layer(h_hbm, a_s_hbm, a_d_hbm, cvec_hbm, src_hbm, dst_hbm, out_hbm,
                 a_s_loc, a_d_loc, den_loc, srcbuf, dstbuf, idxbuf, exbuf,
                 rowbuf, cvec_loc, acc, den_sh, sem):
        c = lax.axis_index("c")
        s = lax.axis_index("s")
        head = c if two_heads else 0

        pltpu.sync_copy(a_s_hbm.at[head], a_s_loc)
        pltpu.sync_copy(a_d_hbm.at[head], a_d_loc)
        pltpu.sync_copy(cvec_hbm.at[head], cvec_loc)
        cv = cvec_loc[...]

        zv = jnp.zeros((16,), jnp.float32)

        def zrow(i, carry):
            for r in range(H // 16):
                rowbuf[i, pl.ds(r * 16, 16)] = zv
            return carry

        lax.fori_loop(0, K, zrow, 0)
        for g in range(K // 16):
            exbuf[pl.ds(g * 16, 16)] = zv
        for b in range(STRIPE // K):
            off = s * STRIPE + b * K
            pltpu.sync_copy(rowbuf, acc.at[pl.ds(off, K)])
            pltpu.sync_copy(exbuf, den_sh.at[pl.ds(off, K)])
        plsc.subcore_barrier()

        nch_t = (NCH - s + NT - 1) // NT

        def den_body(i, carry):
            base = (s + i * NT) * K
            pltpu.sync_copy(src_hbm.at[pl.ds(base, K)], srcbuf)
            pltpu.sync_copy(dst_hbm.at[pl.ds(base, K)], dstbuf)
            _alpha_groups(srcbuf, dstbuf, a_s_loc, a_d_loc, cv, None, exbuf)
            pltpu.sync_copy(exbuf, den_sh.at[dstbuf], add=True)
            return carry

        lax.fori_loop(0, nch_t, den_body, 0)
        plsc.subcore_barrier()
        pltpu.sync_copy(den_sh, den_loc)

        if two_heads:
            msg_base0 = s * K
            nmsg_t = nch_t
        else:
            half = NCH // 2
            msg_base0 = (c * half + s) * K
            nmsg_t = (half - s + NT - 1) // NT

        def msg_body(i, carry):
            base = msg_base0 + i * NT * K
            pltpu.sync_copy(src_hbm.at[pl.ds(base, K)], srcbuf)
            pltpu.sync_copy(dst_hbm.at[pl.ds(base, K)], dstbuf)
            if two_heads:
                cb = lax.broadcast(c, (16,))
                for g in range(K // 16):
                    sl = pl.ds(g * 16, 16)
                    idxbuf[sl] = srcbuf[sl] * 2 + cb
                gidx = idxbuf
            else:
                gidx = srcbuf
            cp = pltpu.async_copy(h_hbm.at[gidx], rowbuf, sem)
            _alpha_groups(srcbuf, dstbuf, a_s_loc, a_d_loc, cv, den_loc, exbuf)
            cp.wait()

            def scale(j, carry2):
                av = plsc.load_gather(exbuf, [lax.broadcast(j, (16,))])
                for r in range(H // 16):
                    sl = pl.ds(r * 16, 16)
                    rowbuf[j, sl] = rowbuf[j, sl] * av
                return carry2

            lax.fori_loop(0, K, scale, 0)
            pltpu.sync_copy(rowbuf, acc.at[dstbuf], add=True)
            return carry

        lax.fori_loop(0, nmsg_t, msg_body, 0)
        plsc.subcore_barrier()

        for b in range(STRIPE // K):
            off = s * STRIPE + b * K
            pltpu.sync_copy(acc.at[pl.ds(off, K)], out_hbm.at[c, pl.ds(off, K)])

    return sc_layer


_sc_cache = {}


def _sc_layer(two_heads):
    if two_heads not in _sc_cache:
        _sc_cache[two_heads] = _make_sc_layer(two_heads)
    return _sc_cache[two_heads]


def _tc1_body(x_ref, w1_ref, m1_ref, h1_ref, asd_ref):
    xb = x_ref[...]
    hb = jnp.dot(xb, w1_ref[...], preferred_element_type=jnp.float32)
    h1_ref[...] = hb
    asd_ref[...] = jnp.dot(hb, m1_ref[...], preferred_element_type=jnp.float32)


def _tc2_body(a_ref, b_ref, b1_ref, w2_ref, m2_ref, h2_ref, asd_ref):
    v = 0.5 * (a_ref[...] + b_ref[...]) + b1_ref[...]
    gb = jnp.where(v > 0, v, jnp.exp(v) - 1.0)
    hb = jnp.dot(gb, w2_ref[...], preferred_element_type=jnp.float32)
    h2_ref[...] = hb
    asd_ref[...] = jnp.dot(hb, m2_ref[...], preferred_element_type=jnp.float32)


def _tc3_body(a_ref, b_ref, b2_ref, batch_ref, wc1_ref, bc1_ref, wc2_ref,
              bc2_ref, out_ref, pool_acc, cnt_acc):
    i = pl.program_id(0)

    @pl.when(i == 0)
    def _():
        pool_acc[...] = jnp.zeros_like(pool_acc)
        cnt_acc[...] = jnp.zeros_like(cnt_acc)

    v = a_ref[...] + b_ref[...] + b2_ref[...]
    hb = jnp.where(v > 0, v, jnp.exp(v) - 1.0)
    bb = batch_ref[...]
    gi = lax.broadcasted_iota(jnp.int32, (1, G), 1)
    oh = (bb == gi).astype(jnp.float32)
    dn = (((0,), (0,)), ((), ()))
    pool_acc[...] += lax.dot_general(oh, hb, dn,
                                     preferred_element_type=jnp.float32)
    cnt_acc[...] += lax.dot_general(oh, jnp.ones_like(hb), dn,
                                    preferred_element_type=jnp.float32)

    @pl.when(i == pl.num_programs(0) - 1)
    def _():
        pooled = pool_acc[...] / jnp.maximum(cnt_acc[...], 1.0)
        z = jnp.dot(pooled, wc1_ref[...], preferred_element_type=jnp.float32)
        z = jnp.maximum(z + bc1_ref[...], 0.0)
        out_ref[...] = jnp.dot(z, wc2_ref[...],
                               preferred_element_type=jnp.float32) + bc2_ref[...]


def _full_spec(shape):
    return pl.BlockSpec(shape, lambda i: tuple(0 for _ in shape))


def _row_spec(cols):
    return pl.BlockSpec((BLK, cols), lambda i: (i, 0))


def kernel(x, edge_index, batch, W1, att_src1, att_dst1, b1, W2, att_src2,
           att_dst2, b2, Wc1, bc1, Wc2, bc2):
    src = edge_index[0]
    dst = edge_index[1]

    x_pad = jnp.pad(x, ((0, NP - N), (0, 0)))
    batch_pad = jnp.pad(batch, (0, NP - N), constant_values=G).reshape(NP, 1)

    M1 = jnp.zeros((2 * H, 128), jnp.float32)
    M1 = M1.at[0:H, 0].set(att_src1[0]).at[H:2 * H, 1].set(att_src1[1])
    M1 = M1.at[0:H, 2].set(att_dst1[0]).at[H:2 * H, 3].set(att_dst1[1])
    M2 = jnp.zeros((H, 128), jnp.float32)
    M2 = M2.at[:, 0].set(att_src2[0]).at[:, 1].set(att_dst2[0])

    h1, asd1 = pl.pallas_call(
        _tc1_body,
        grid=(NP // BLK,),
        in_specs=[_row_spec(D), _full_spec((D, 2 * H)), _full_spec((2 * H, 128))],
        out_specs=[_row_spec(2 * H), _row_spec(128)],
        out_shape=[jax.ShapeDtypeStruct((NP, 2 * H), jnp.float32),
                   jax.ShapeDtypeStruct((NP, 128), jnp.float32)],
    )(x_pad, W1, M1)

    a_s1 = jnp.stack([asd1[:, 0], asd1[:, 1]])
    a_d1 = jnp.stack([asd1[:, 2], asd1[:, 3]])
    c1 = jnp.maximum(jnp.max(a_s1, axis=1) + jnp.max(a_d1, axis=1), 0.0)
    cvec1 = jnp.broadcast_to(c1[:, None], (2, 16))

    h1v = h1.reshape(2 * NP, H)
    out1 = _sc_layer(True)(h1v, a_s1, a_d1, cvec1, src, dst)

    h2, asd2 = pl.pallas_call(
        _tc2_body,
        grid=(NP // BLK,),
        in_specs=[_row_spec(H), _row_spec(H), _full_spec((1, H)),
                  _full_spec((H, H)), _full_spec((H, 128))],
        out_specs=[_row_spec(H), _row_spec(128)],
        out_shape=[jax.ShapeDtypeStruct((NP, H), jnp.float32),
                   jax.ShapeDtypeStruct((NP, 128), jnp.float32)],
    )(out1[0], out1[1], b1.reshape(1, H), W2, M2)

    a_s2 = asd2[:, 0].reshape(1, NP)
    a_d2 = asd2[:, 1].reshape(1, NP)
    c2 = jnp.maximum(jnp.max(a_s2) + jnp.max(a_d2), 0.0)
    cvec2 = jnp.broadcast_to(c2, (1, 16))

    out2 = _sc_layer(False)(h2, a_s2, a_d2, cvec2, src, dst)

    out3 = pl.pallas_call(
        _tc3_body,
        grid=(NP // BLK,),
        in_specs=[_row_spec(H), _row_spec(H), _full_spec((1, H)),
                  pl.BlockSpec((BLK, 1), lambda i: (i, 0)),
                  _full_spec((H, H)), _full_spec((1, H)),
                  _full_spec((H, 128)), _full_spec((1, 128))],
        out_specs=pl.BlockSpec((G, 128), lambda i: (0, 0)),
        out_shape=jax.ShapeDtypeStruct((G, 128), jnp.float32),
        scratch_shapes=[pltpu.VMEM((G, 128), jnp.float32),
                        pltpu.VMEM((G, 128), jnp.float32)],
    )(out2[0], out2[1], b2.reshape(1, H), batch_pad, Wc1,
      bc1.reshape(1, H), jnp.pad(Wc2, ((0, 0), (0, 127))),
      jnp.broadcast_to(bc2.reshape(1, 1), (1, 128)))

    return out3[:, 0]
